# baseline (speedup 1.0000x reference)
import functools

import jax
import jax.numpy as jnp
from jax import lax
from jax.experimental import pallas as pl
from jax.experimental.pallas import tpu as pltpu
from jax.experimental.pallas import tpu_sc as plsc

HIDDEN = 16
EDIM = HIDDEN * 4
NC, NS, L = 2, 16, 16
NW = NC * NS


def _relnorm_body(r0, r1, r2, r3, y0, y1, y2, y3):
    a, b, c, d = r0[...], r1[...], r2[...], r3[...]
    n2 = a * a + b * b + c * c + d * d
    rinv = jnp.minimum(lax.rsqrt(n2), 1e12)
    y0[...] = a * rinv
    y1[...] = b * rinv
    y2[...] = c * rinv
    y3[...] = d * rinv


def _normalize_relation_planar(relation_embedding):
    nrel = relation_embedding.shape[0]
    r = relation_embedding.reshape(nrel, HIDDEN, 4)
    comps = [r[:, :, c] for c in range(4)]
    outs = pl.pallas_call(
        _relnorm_body,
        out_shape=[jax.ShapeDtypeStruct((nrel, HIDDEN), jnp.float32)] * 4,
    )(*comps)
    return jnp.concatenate(outs, axis=1)


def _make_sc_scorer(batch, nentity, nrel):
    bpw = batch // NW
    ngroups = bpw // L

    mesh = plsc.VectorSubcoreMesh(core_axis_name="c", subcore_axis_name="s")

    @functools.partial(
        pl.kernel,
        out_type=jax.ShapeDtypeStruct((batch,), jnp.float32),
        mesh=mesh,
        compiler_params=pltpu.CompilerParams(
            needs_layout_passes=False, use_tc_tiling_on_sc=False
        ),
        scratch_types=[
            pltpu.VMEM((bpw,), jnp.int32),
            pltpu.VMEM((bpw,), jnp.int32),
            pltpu.VMEM((bpw,), jnp.int32),
            pltpu.VMEM((bpw, EDIM), jnp.float32),
            pltpu.VMEM((bpw, EDIM), jnp.float32),
            pltpu.VMEM((bpw, EDIM), jnp.float32),
            pltpu.VMEM((bpw,), jnp.float32),
            pltpu.SemaphoreType.DMA,
        ],
    )
    def scorer(heads_hbm, rels_hbm, tails_hbm, ent_hbm, reln_hbm, phi_hbm,
               idx_h, idx_r, idx_t, rows_h, rows_r, rows_t, phi_v, sem):
        wid = lax.axis_index("s") * NC + lax.axis_index("c")
        base = wid * bpw

        pltpu.sync_copy(heads_hbm.at[pl.ds(base, bpw)], idx_h)
        pltpu.sync_copy(rels_hbm.at[pl.ds(base, bpw)], idx_r)
        pltpu.sync_copy(tails_hbm.at[pl.ds(base, bpw)], idx_t)

        ch = pltpu.async_copy(ent_hbm.at[idx_h], rows_h, sem)
        cr = pltpu.async_copy(reln_hbm.at[idx_r], rows_r, sem)
        ct = pltpu.async_copy(ent_hbm.at[idx_t], rows_t, sem)
        ch.wait()
        cr.wait()
        ct.wait()

        iota = lax.broadcasted_iota(jnp.int32, (L,), 0)

        def group_body(g, carry):
            rowv = g * L + iota
            acc = jnp.zeros((L,), jnp.float32)
            for k in range(HIDDEN):
                hw = plsc.load_gather(rows_h, [rowv, jnp.full((L,), 4 * k + 0, jnp.int32)])
                hx = plsc.load_gather(rows_h, [rowv, jnp.full((L,), 4 * k + 1, jnp.int32)])
                hy = plsc.load_gather(rows_h, [rowv, jnp.full((L,), 4 * k + 2, jnp.int32)])
                hz = plsc.load_gather(rows_h, [rowv, jnp.full((L,), 4 * k + 3, jnp.int32)])
                tw = plsc.load_gather(rows_t, [rowv, jnp.full((L,), 4 * k + 0, jnp.int32)])
                tx = plsc.load_gather(rows_t, [rowv, jnp.full((L,), 4 * k + 1, jnp.int32)])
                ty = plsc.load_gather(rows_t, [rowv, jnp.full((L,), 4 * k + 2, jnp.int32)])
                tz = plsc.load_gather(rows_t, [rowv, jnp.full((L,), 4 * k + 3, jnp.int32)])
                rw = plsc.load_gather(rows_r, [rowv, jnp.full((L,), k, jnp.int32)])
                rx = plsc.load_gather(rows_r, [rowv, jnp.full((L,), HIDDEN + k, jnp.int32)])
                ry = plsc.load_gather(rows_r, [rowv, jnp.full((L,), 2 * HIDDEN + k, jnp.int32)])
                rz = plsc.load_gather(rows_r, [rowv, jnp.full((L,), 3 * HIDDEN + k, jnp.int32)])

                p = hw * rw - hx * rx - hy * ry - hz * rz
                q = hw * rx + hx * rw + hy * rz - hz * ry
                u = hw * ry - hx * rz + hy * rw + hz * rx
                v = hw * rz + hx * ry - hy * rx + hz * rw
                acc = acc + (p * tw + q * tx + u * ty + v * tz)
            phi_v[pl.ds(g * L, L)] = acc
            return carry

        lax.fori_loop(0, ngroups, group_body, 0)
        pltpu.sync_copy(phi_v, phi_hbm.at[pl.ds(base, bpw)])

    return scorer


def _loss_body(phi, y, out):
    out[...] = jnp.log(1.0 + jnp.exp(-y[...] * phi[...]))


def _loss(phi, Y):
    batch = phi.shape[0]
    r = batch // 128
    out = pl.pallas_call(
        _loss_body,
        out_shape=jax.ShapeDtypeStruct((r, 128), jnp.float32),
    )(phi.reshape(r, 128), Y.reshape(r, 128))
    return out.reshape(batch)


@jax.jit
def kernel(sample, Y, entity_embedding, relation_embedding):
    batch = sample.shape[0]
    nentity = entity_embedding.shape[0]
    nrel = relation_embedding.shape[0]

    reln_planar = _normalize_relation_planar(relation_embedding)

    heads = sample[:, 0].astype(jnp.int32)
    rels = sample[:, 1].astype(jnp.int32)
    tails = sample[:, 2].astype(jnp.int32)

    scorer = _make_sc_scorer(batch, nentity, nrel)
    phi = scorer(heads, rels, tails, entity_embedding, reln_planar)

    loss = _loss(phi, Y)
    return (loss, Y)
